# baseline (speedup 1.0000x reference)
import functools

import jax
import jax.numpy as jnp
from jax import lax
from jax.experimental import pallas as pl
from jax.experimental.pallas import tpu as pltpu
from jax.experimental.pallas import tpu_sc as plsc

B = 16384
D = 64
N = 1000001

_info = plsc.get_sparse_core_info()
_NC = _info.num_cores
_NS = _info.num_subcores
_NW = _NC * _NS

CHUNK = 128
NCHUNK = 7812
TAIL_START = NCHUNK * CHUNK
NBUCKET = 246
CAPC = 48
NRING = 10
OUT_ROWS = B + _NW * 16

_mesh = plsc.VectorSubcoreMesh(core_axis_name="c", subcore_axis_name="s")


@functools.partial(
    pl.kernel,
    mesh=_mesh,
    out_type=jax.ShapeDtypeStruct((OUT_ROWS, 128), jnp.float32),
    scratch_types=[
        pltpu.VMEM((B,), jnp.int32),
        pltpu.VMEM((NRING, D, CHUNK), jnp.float32),
        pltpu.VMEM((B + 16,), jnp.int32),
        pltpu.VMEM((NBUCKET * CAPC + 16,), jnp.int32),
        pltpu.VMEM((16, 128), jnp.float32),
        pltpu.VMEM((16,), jnp.int32),
        pltpu.SMEM((NBUCKET,), jnp.int32),
        pltpu.SMEM((4,), jnp.int32),
        pltpu.SemaphoreType.DMA,
        pltpu.SemaphoreType.DMA,
        pltpu.SemaphoreType.DMA,
        pltpu.SemaphoreType.DMA,
        pltpu.SemaphoreType.DMA,
        pltpu.SemaphoreType.DMA,
        pltpu.SemaphoreType.DMA,
        pltpu.SemaphoreType.DMA,
        pltpu.SemaphoreType.DMA,
        pltpu.SemaphoreType.DMA,
        pltpu.SemaphoreType.DMA,
    ],
    compiler_params=pltpu.CompilerParams(
        use_tc_tiling_on_sc=True, needs_layout_passes=False),
)
def _embed_stream(idx_hbm, table_t_hbm, out_hbm, idx_v, ring_v, flat_v,
                  buck_v, stage_v, bvec_v, counts_s, misc_s,
                  sem_i, sem0, sem1, sem2, sem3, sem4, sem5, sem6, sem7, sem8, sem9):
    sems = (sem0, sem1, sem2, sem3, sem4, sem5, sem6, sem7, sem8, sem9)
    wid = lax.axis_index("s") * _NC + lax.axis_index("c")
    cstart = (wid * NCHUNK) // _NW
    cend = ((wid + 1) * NCHUNK) // _NW
    n = cend - cstart
    l0 = cstart * CHUNK
    l1 = cend * CHUNK
    iota = lax.iota(jnp.int32, 16)
    trash_vec = B + wid * 16 + iota

    idx_cp = pltpu.async_copy(idx_hbm, idx_v, sem_i)
    for r in range(NRING):
        @pl.when(r < n)
        def _(r=r):
            pltpu.async_copy(
                table_t_hbm.at[:, pl.ds((cstart + r) * CHUNK, CHUNK)],
                ring_v.at[r], sems[r])

    idx_cp.wait()

    l1_eff = jnp.where(wid == _NW - 1, jnp.int32(2**30), jnp.int32(l1))

    def scan_body(v, cnt):
        x = idx_v[pl.ds(v * 16, 16)]
        m = (x >= l0) & (x < l1_eff)
        pos = x - l0
        packed = (pos << 14) | (v * 16 + iota)
        key = jnp.where(m, jnp.int32(0), jnp.int32(1))
        _, sv = plsc.sort_key_val(key, packed)
        flat_v[pl.ds(cnt, 16)] = sv
        return cnt + jnp.sum(jnp.where(m, 1, 0))

    nhits = lax.fori_loop(0, B // 16, scan_body, jnp.int32(0))

    def zero_body(i, _):
        counts_s[i] = 0
        return ()

    lax.fori_loop(0, NBUCKET, zero_body, ())

    def bucket_body(h, _):
        p = flat_v[pl.ds(h, 16)][0]
        qr = p >> (14 + 7)
        c = counts_s[qr]

        @pl.when(c < CAPC)
        def _():
            at = qr * CAPC + c
            off = at & ~jnp.int32(15)
            lane = at & 15
            vec = buck_v[pl.ds(off, 16)]
            buck_v[pl.ds(off, 16)] = jnp.where(iota == lane, p, vec)
            counts_s[qr] = c + 1

        return ()

    lax.fori_loop(0, nhits, bucket_body, ())
    misc_s[0] = 0
    bvec_v[...] = trash_vec

    def extract(bufref, qr):
        cnt = counts_s[qr]

        def hit_body(h, _):
            p = buck_v[pl.ds(qr * CAPC + h, 16)][0]
            b = p & jnp.int32(16383)
            l = (p >> 14) - qr * CHUNK
            lvec = jnp.broadcast_to(l, (16,))
            slot = misc_s[0]
            for k in range(4):
                col = plsc.load_gather(bufref, [iota + 16 * k, lvec])
                stage_v[slot, pl.ds(16 * k, 16)] = col
            bvec = bvec_v[...]
            bvec_v[...] = jnp.where(iota == slot, b, bvec)

            @pl.when(slot == 15)
            def _():
                pltpu.sync_copy(stage_v, out_hbm.at[bvec_v[...]])
                bvec_v[...] = trash_vec

            misc_s[0] = jnp.where(slot == 15, 0, slot + 1)
            return ()

        lax.fori_loop(0, cnt, hit_body, ())

    def ring_body(it, _):
        q4 = cstart + NRING * it
        for r in range(NRING):
            q = q4 + r
            pltpu.make_async_copy(
                table_t_hbm.at[:, pl.ds(q * CHUNK, CHUNK)],
                ring_v.at[r], sems[r]).wait()
            extract(ring_v.at[r], q - cstart)

            @pl.when(q + NRING < cend)
            def _(r=r, q=q):
                pltpu.async_copy(
                    table_t_hbm.at[:, pl.ds((q + NRING) * CHUNK, CHUNK)],
                    ring_v.at[r], sems[r])

        return ()

    lax.fori_loop(0, n // NRING, ring_body, ())

    for r in range(NRING - 1):
        q_off = (n // NRING) * NRING + r

        @pl.when(q_off < n)
        def _(r=r, q_off=q_off):
            pltpu.make_async_copy(
                table_t_hbm.at[:, pl.ds((cstart + q_off) * CHUNK, CHUNK)],
                ring_v.at[r], sems[r]).wait()
            extract(ring_v.at[r], q_off)

    pltpu.sync_copy(stage_v, out_hbm.at[bvec_v[...]])


def kernel(batch, table):
    idx = batch.astype(jnp.int32)
    tt = table.T
    kout = _embed_stream(idx, tt)
    main = kout[:B, :D]
    tail_small = lax.slice(table, (TAIL_START, 0), (N, D))
    t_idx = jnp.clip(idx - TAIL_START, 0, N - TAIL_START - 1)
    t_out = jnp.take(tail_small, t_idx, axis=0)
    sel = (idx >= TAIL_START)[:, None]
    return jnp.where(sel, t_out, main).reshape(B, 1, D)


# speedup vs baseline: 1.0081x; 1.0081x over previous
import functools

import jax
import jax.numpy as jnp
from jax import lax
from jax.experimental import pallas as pl
from jax.experimental.pallas import tpu as pltpu
from jax.experimental.pallas import tpu_sc as plsc

B = 16384
D = 64
N = 1000001

_info = plsc.get_sparse_core_info()
_NC = _info.num_cores
_NS = _info.num_subcores
_NW = _NC * _NS

CHUNK = 256
NCHUNK = 3906
TAIL_START = NCHUNK * CHUNK
NBUCKET = 124
CAPC = 64
NRING = 5
OUT_ROWS = B + _NW * 16

_mesh = plsc.VectorSubcoreMesh(core_axis_name="c", subcore_axis_name="s")


@functools.partial(
    pl.kernel,
    mesh=_mesh,
    out_type=jax.ShapeDtypeStruct((OUT_ROWS, 128), jnp.float32),
    scratch_types=[
        pltpu.VMEM((B,), jnp.int32),
        pltpu.VMEM((NRING, D, CHUNK), jnp.float32),
        pltpu.VMEM((B + 16,), jnp.int32),
        pltpu.VMEM((NBUCKET * CAPC + 16,), jnp.int32),
        pltpu.VMEM((16, 128), jnp.float32),
        pltpu.VMEM((16,), jnp.int32),
        pltpu.SMEM((NBUCKET,), jnp.int32),
        pltpu.SMEM((4,), jnp.int32),
        pltpu.SemaphoreType.DMA,
        pltpu.SemaphoreType.DMA,
        pltpu.SemaphoreType.DMA,
        pltpu.SemaphoreType.DMA,
        pltpu.SemaphoreType.DMA,
        pltpu.SemaphoreType.DMA,
    ],
    compiler_params=pltpu.CompilerParams(
        use_tc_tiling_on_sc=True, needs_layout_passes=False),
)
def _embed_stream(idx_hbm, table_t_hbm, out_hbm, idx_v, ring_v, flat_v,
                  buck_v, stage_v, bvec_v, counts_s, misc_s,
                  sem_i, sem0, sem1, sem2, sem3, sem4):
    sems = (sem0, sem1, sem2, sem3, sem4)
    wid = lax.axis_index("s") * _NC + lax.axis_index("c")
    cstart = (wid * NCHUNK) // _NW
    cend = ((wid + 1) * NCHUNK) // _NW
    n = cend - cstart
    l0 = cstart * CHUNK
    l1 = cend * CHUNK
    iota = lax.iota(jnp.int32, 16)
    trash_vec = B + wid * 16 + iota

    idx_cp = pltpu.async_copy(idx_hbm, idx_v, sem_i)
    for r in range(NRING):
        @pl.when(r < n)
        def _(r=r):
            pltpu.async_copy(
                table_t_hbm.at[:, pl.ds((cstart + r) * CHUNK, CHUNK)],
                ring_v.at[r], sems[r])

    idx_cp.wait()

    l1_eff = jnp.where(wid == _NW - 1, jnp.int32(2**30), jnp.int32(l1))

    def scan_body(v, cnt):
        x = idx_v[pl.ds(v * 16, 16)]
        m = (x >= l0) & (x < l1_eff)
        pos = x - l0
        packed = (pos << 14) | (v * 16 + iota)
        key = jnp.where(m, jnp.int32(0), jnp.int32(1))
        _, sv = plsc.sort_key_val(key, packed)
        flat_v[pl.ds(cnt, 16)] = sv
        return cnt + jnp.sum(jnp.where(m, 1, 0))

    nhits = lax.fori_loop(0, B // 16, scan_body, jnp.int32(0))

    def zero_body(i, _):
        counts_s[i] = 0
        return ()

    lax.fori_loop(0, NBUCKET, zero_body, ())

    def bucket_body(h, _):
        p = flat_v[pl.ds(h, 16)][0]
        qr = p >> (14 + 8)
        c = counts_s[qr]

        @pl.when(c < CAPC)
        def _():
            at = qr * CAPC + c
            off = at & ~jnp.int32(15)
            lane = at & 15
            vec = buck_v[pl.ds(off, 16)]
            buck_v[pl.ds(off, 16)] = jnp.where(iota == lane, p, vec)
            counts_s[qr] = c + 1

        return ()

    lax.fori_loop(0, nhits, bucket_body, ())
    misc_s[0] = 0
    bvec_v[...] = trash_vec

    def extract(bufref, qr):
        cnt = counts_s[qr]

        def hit_body(h, _):
            p = buck_v[pl.ds(qr * CAPC + h, 16)][0]
            b = p & jnp.int32(16383)
            l = (p >> 14) - qr * CHUNK
            lvec = jnp.broadcast_to(l, (16,))
            slot = misc_s[0]
            for k in range(4):
                col = plsc.load_gather(bufref, [iota + 16 * k, lvec])
                stage_v[slot, pl.ds(16 * k, 16)] = col
            bvec = bvec_v[...]
            bvec_v[...] = jnp.where(iota == slot, b, bvec)

            @pl.when(slot == 15)
            def _():
                pltpu.sync_copy(stage_v, out_hbm.at[bvec_v[...]])
                bvec_v[...] = trash_vec

            misc_s[0] = jnp.where(slot == 15, 0, slot + 1)
            return ()

        lax.fori_loop(0, cnt, hit_body, ())

    def ring_body(it, _):
        q4 = cstart + NRING * it
        for r in range(NRING):
            q = q4 + r
            pltpu.make_async_copy(
                table_t_hbm.at[:, pl.ds(q * CHUNK, CHUNK)],
                ring_v.at[r], sems[r]).wait()
            extract(ring_v.at[r], q - cstart)

            @pl.when(q + NRING < cend)
            def _(r=r, q=q):
                pltpu.async_copy(
                    table_t_hbm.at[:, pl.ds((q + NRING) * CHUNK, CHUNK)],
                    ring_v.at[r], sems[r])

        return ()

    lax.fori_loop(0, n // NRING, ring_body, ())

    for r in range(NRING - 1):
        q_off = (n // NRING) * NRING + r

        @pl.when(q_off < n)
        def _(r=r, q_off=q_off):
            pltpu.make_async_copy(
                table_t_hbm.at[:, pl.ds((cstart + q_off) * CHUNK, CHUNK)],
                ring_v.at[r], sems[r]).wait()
            extract(ring_v.at[r], q_off)

    pltpu.sync_copy(stage_v, out_hbm.at[bvec_v[...]])


def kernel(batch, table):
    idx = batch.astype(jnp.int32)
    tt = table.T
    kout = _embed_stream(idx, tt)
    main = kout[:B, :D]
    tail_small = lax.slice(table, (TAIL_START, 0), (N, D))
    t_idx = jnp.clip(idx - TAIL_START, 0, N - TAIL_START - 1)
    t_out = jnp.take(tail_small, t_idx, axis=0)
    sel = (idx >= TAIL_START)[:, None]
    return jnp.where(sel, t_out, main).reshape(B, 1, D)


# speedup vs baseline: 1.0365x; 1.0282x over previous
import functools

import jax
import jax.numpy as jnp
from jax import lax
from jax.experimental import pallas as pl
from jax.experimental.pallas import tpu as pltpu
from jax.experimental.pallas import tpu_sc as plsc

B = 16384
D = 64
N = 1000001

_info = plsc.get_sparse_core_info()
_NC = _info.num_cores
_NS = _info.num_subcores
_NW = _NC * _NS

CHUNK = 256
NCHUNK = 3906
TAIL_START = NCHUNK * CHUNK
NBUCKET = 124
CAPC = 64
NRING = 5
OUT_ROWS = B + _NW * 16

_mesh = plsc.VectorSubcoreMesh(core_axis_name="c", subcore_axis_name="s")


@functools.partial(
    pl.kernel,
    mesh=_mesh,
    out_type=jax.ShapeDtypeStruct((OUT_ROWS, 128), jnp.float32),
    scratch_types=[
        pltpu.VMEM((B,), jnp.int32),
        pltpu.VMEM((NRING, D, CHUNK), jnp.float32),
        pltpu.VMEM((B + 16,), jnp.int32),
        pltpu.VMEM((NBUCKET * CAPC + 16,), jnp.int32),
        pltpu.VMEM((16, 128), jnp.float32),
        pltpu.VMEM((16,), jnp.int32),
        pltpu.SMEM((NBUCKET,), jnp.int32),
        pltpu.SMEM((4,), jnp.int32),
        pltpu.SemaphoreType.DMA,
        pltpu.SemaphoreType.DMA,
        pltpu.SemaphoreType.DMA,
        pltpu.SemaphoreType.DMA,
        pltpu.SemaphoreType.DMA,
        pltpu.SemaphoreType.DMA,
    ],
    compiler_params=pltpu.CompilerParams(
        use_tc_tiling_on_sc=True, needs_layout_passes=False),
)
def _embed_stream(idx_hbm, table_t_hbm, tail_hbm, out_hbm, idx_v, ring_v, flat_v,
                  buck_v, stage_v, bvec_v, counts_s, misc_s,
                  sem_i, sem0, sem1, sem2, sem3, sem4):
    sems = (sem0, sem1, sem2, sem3, sem4)
    wid = lax.axis_index("s") * _NC + lax.axis_index("c")
    cstart = (wid * NCHUNK) // _NW
    cend = ((wid + 1) * NCHUNK) // _NW
    n = cend - cstart
    l0 = cstart * CHUNK
    l1 = cend * CHUNK
    iota = lax.iota(jnp.int32, 16)
    trash_vec = B + wid * 16 + iota

    idx_cp = pltpu.async_copy(idx_hbm, idx_v, sem_i)
    for r in range(NRING):
        @pl.when(r < n)
        def _(r=r):
            pltpu.async_copy(
                table_t_hbm.at[:, pl.ds((cstart + r) * CHUNK, CHUNK)],
                ring_v.at[r], sems[r])

    idx_cp.wait()

    l1_eff = jnp.where(wid == _NW - 1, jnp.int32(2**30), jnp.int32(l1))

    def scan_body(v, cnt):
        x = idx_v[pl.ds(v * 16, 16)]
        m = (x >= l0) & (x < l1_eff)
        pos = x - l0
        packed = (pos << 14) | (v * 16 + iota)
        key = jnp.where(m, jnp.int32(0), jnp.int32(1))
        _, sv = plsc.sort_key_val(key, packed)
        flat_v[pl.ds(cnt, 16)] = sv
        return cnt + jnp.sum(jnp.where(m, 1, 0))

    nhits = lax.fori_loop(0, B // 16, scan_body, jnp.int32(0))

    def zero_body(i, _):
        counts_s[i] = 0
        return ()

    lax.fori_loop(0, NBUCKET, zero_body, ())

    def bucket_body(h, _):
        p = flat_v[pl.ds(h, 16)][0]
        qr = p >> (14 + 8)
        c = counts_s[qr]

        @pl.when(c < CAPC)
        def _():
            at = qr * CAPC + c
            off = at & ~jnp.int32(15)
            lane = at & 15
            vec = buck_v[pl.ds(off, 16)]
            buck_v[pl.ds(off, 16)] = jnp.where(iota == lane, p, vec)
            counts_s[qr] = c + 1

        return ()

    lax.fori_loop(0, nhits, bucket_body, ())
    misc_s[0] = 0
    bvec_v[...] = trash_vec

    def extract(bufref, qr):
        cnt = counts_s[qr]

        def hit_body(h, _):
            p = buck_v[pl.ds(qr * CAPC + h, 16)][0]
            b = p & jnp.int32(16383)
            l = (p >> 14) - qr * CHUNK
            lvec = jnp.broadcast_to(l, (16,))
            slot = misc_s[0]
            for k in range(4):
                col = plsc.load_gather(bufref, [iota + 16 * k, lvec])
                stage_v[slot, pl.ds(16 * k, 16)] = col
            bvec = bvec_v[...]
            bvec_v[...] = jnp.where(iota == slot, b, bvec)

            @pl.when(slot == 15)
            def _():
                pltpu.sync_copy(stage_v, out_hbm.at[bvec_v[...]])
                bvec_v[...] = trash_vec

            misc_s[0] = jnp.where(slot == 15, 0, slot + 1)
            return ()

        lax.fori_loop(0, cnt, hit_body, ())

    def ring_body(it, _):
        q4 = cstart + NRING * it
        for r in range(NRING):
            q = q4 + r
            pltpu.make_async_copy(
                table_t_hbm.at[:, pl.ds(q * CHUNK, CHUNK)],
                ring_v.at[r], sems[r]).wait()
            extract(ring_v.at[r], q - cstart)

            @pl.when(q + NRING < cend)
            def _(r=r, q=q):
                pltpu.async_copy(
                    table_t_hbm.at[:, pl.ds((q + NRING) * CHUNK, CHUNK)],
                    ring_v.at[r], sems[r])

        return ()

    lax.fori_loop(0, n // NRING, ring_body, ())

    for r in range(NRING - 1):
        q_off = (n // NRING) * NRING + r

        @pl.when(q_off < n)
        def _(r=r, q_off=q_off):
            pltpu.make_async_copy(
                table_t_hbm.at[:, pl.ds((cstart + q_off) * CHUNK, CHUNK)],
                ring_v.at[r], sems[r]).wait()
            extract(ring_v.at[r], q_off)

    @pl.when(wid == _NW - 1)
    def _():
        pltpu.sync_copy(tail_hbm, ring_v.at[0, :, pl.ds(0, 128)])
        extract(ring_v.at[0], n)

    pltpu.sync_copy(stage_v, out_hbm.at[bvec_v[...]])


def kernel(batch, table):
    idx = batch.astype(jnp.int32)
    tt = table.T
    tail_tt = jnp.pad(
        lax.slice(tt, (0, TAIL_START), (D, N)),
        ((0, 0), (0, 128 - (N - TAIL_START))))
    kout = _embed_stream(idx, tt, tail_tt)
    return kout[:B, :D].reshape(B, 1, D)
